# baseline (speedup 1.0000x reference)
import functools

import jax
import jax.numpy as jnp
from jax import lax
from jax.experimental import pallas as pl
from jax.experimental.pallas import tpu as pltpu
from jax.experimental.pallas import tpu_sc as plsc

_EPS = 0.1
_V = 32000
_N = 2048
_IGNORE = 0
_CONF = 1.0 - _EPS
_EPS_U = _EPS / (_V - 1)

_BR = 128
_BV = 3200
_NR = _N // _BR
_NV = _V // _BV

_NC = 2
_NS = 16
_NW = _NC * _NS
_BPW = _N // _NW
_L = 16


def _sc_gather_kernel(flat_ref, tgt_ref, out_ref, tgt_v, idx_v, g_v, sem):
    wid = lax.axis_index("s") * _NC + lax.axis_index("c")
    base = wid * _BPW
    pltpu.sync_copy(tgt_ref.at[pl.ds(base, _BPW)], tgt_v)
    for k in range(_BPW // _L):
        t16 = tgt_v[pl.ds(k * _L, _L)]
        rows = (base + k * _L) + lax.iota(jnp.int32, _L)
        idx_v[pl.ds(k * _L, _L)] = rows * _V + t16
    pltpu.async_copy(flat_ref.at[idx_v], g_v, sem).wait()
    pltpu.sync_copy(g_v, out_ref.at[pl.ds(base, _BPW)])


_sc_gather = functools.partial(
    pl.kernel,
    out_type=jax.ShapeDtypeStruct((_N,), jnp.float32),
    mesh=plsc.VectorSubcoreMesh(core_axis_name="c", subcore_axis_name="s"),
    scratch_types=[
        pltpu.VMEM((_BPW,), jnp.int32),
        pltpu.VMEM((_BPW,), jnp.int32),
        pltpu.VMEM((_BPW,), jnp.float32),
        pltpu.SemaphoreType.DMA,
    ],
)(_sc_gather_kernel)


def _loss_kernel(x_ref, tgt_ref, g_ref, out_ref, m_ref, s_ref, t_ref):
    i = pl.program_id(0)
    j = pl.program_id(1)

    x = x_ref[...]

    @pl.when(j == 0)
    def _init():
        m_ref[...] = jnp.full((_BR, 1), -jnp.inf, jnp.float32)
        s_ref[...] = jnp.zeros((_BR, 1), jnp.float32)
        t_ref[...] = jnp.zeros((_BR, 1), jnp.float32)

    m_old = m_ref[...]
    blk_max = jnp.max(x, axis=1, keepdims=True)
    m_new = jnp.maximum(m_old, blk_max)
    s_ref[...] = (s_ref[...] * jnp.exp(m_old - m_new)
                  + jnp.sum(jnp.exp(x - m_new), axis=1, keepdims=True))
    m_ref[...] = m_new
    t_ref[...] = t_ref[...] + jnp.sum(x, axis=1, keepdims=True)

    @pl.when(j == _NV - 1)
    def _finish():
        tgt = tgt_ref[i]
        g = g_ref[i]
        lse = m_ref[...] + jnp.log(s_ref[...])
        gp = g - lse
        loss_rows = (_EPS_U * (_V * lse - t_ref[...])
                     - (_CONF - _EPS_U) * gp)
        loss_rows = jnp.where(tgt == _IGNORE, 0.0, loss_rows)
        part = jnp.sum(loss_rows) * (1.0 / _N)

        @pl.when(i == 0)
        def _first():
            out_ref[0, 0] = part

        @pl.when(i > 0)
        def _rest():
            out_ref[0, 0] = out_ref[0, 0] + part


_tc_loss = pl.pallas_call(
    _loss_kernel,
    grid=(_NR, _NV),
    in_specs=[
        pl.BlockSpec((_BR, _BV), lambda i, j: (i, j)),
        pl.BlockSpec((_NR, _BR, 1), lambda i, j: (0, 0, 0)),
        pl.BlockSpec((_NR, _BR, 1), lambda i, j: (0, 0, 0)),
    ],
    out_specs=pl.BlockSpec((1, 1), lambda i, j: (0, 0),
                           memory_space=pltpu.SMEM),
    out_shape=jax.ShapeDtypeStruct((1, 1), jnp.float32),
    scratch_shapes=[
        pltpu.VMEM((_BR, 1), jnp.float32),
        pltpu.VMEM((_BR, 1), jnp.float32),
        pltpu.VMEM((_BR, 1), jnp.float32),
    ],
)


@jax.jit
def kernel(output, target):
    g = _sc_gather(output.reshape(_N * _V), target)
    out = _tc_loss(output,
                   target.reshape(_NR, _BR, 1),
                   g.reshape(_NR, _BR, 1))
    return out[0, 0]


# speedup vs baseline: 3.1061x; 3.1061x over previous
import functools

import jax
import jax.numpy as jnp
from jax.experimental import pallas as pl
from jax.experimental.pallas import tpu as pltpu

_EPS = 0.1
_V = 32000
_N = 2048
_IGNORE = 0
_CONF = 1.0 - _EPS
_EPS_U = _EPS / (_V - 1)

_BR = 256
_BV = 6400
_NR = _N // _BR
_NV = _V // _BV


def _loss_kernel(x_ref, tgt_ref, out_ref, m_ref, s_ref, t_ref, g_ref):
    i = pl.program_id(0)
    j = pl.program_id(1)

    x = x_ref[...]

    @pl.when(j == 0)
    def _init():
        m_ref[...] = jnp.full((_BR, 1), -jnp.inf, jnp.float32)
        s_ref[...] = jnp.zeros((_BR, 1), jnp.float32)
        t_ref[...] = jnp.zeros((_BR, 1), jnp.float32)
        g_ref[...] = jnp.zeros((_BR, 1), jnp.float32)

    m_old = m_ref[...]
    blk_max = jnp.max(x, axis=1, keepdims=True)
    m_new = jnp.maximum(m_old, blk_max)
    s_ref[...] = (s_ref[...] * jnp.exp(m_old - m_new)
                  + jnp.sum(jnp.exp(x - m_new), axis=1, keepdims=True))
    m_ref[...] = m_new
    t_ref[...] = t_ref[...] + jnp.sum(x, axis=1, keepdims=True)

    tgt = tgt_ref[i]
    cols = jax.lax.broadcasted_iota(jnp.int32, (_BR, _BV), 1) + j * _BV
    g_ref[...] = g_ref[...] + jnp.sum(
        jnp.where(cols == tgt, x, 0.0), axis=1, keepdims=True)

    @pl.when(j == _NV - 1)
    def _finish():
        lse = m_ref[...] + jnp.log(s_ref[...])
        gp = g_ref[...] - lse
        loss_rows = (_EPS_U * (_V * lse - t_ref[...])
                     - (_CONF - _EPS_U) * gp)
        loss_rows = jnp.where(tgt == _IGNORE, 0.0, loss_rows)
        part = jnp.sum(loss_rows) * (1.0 / _N)

        @pl.when(i == 0)
        def _first():
            out_ref[0, 0] = part

        @pl.when(i > 0)
        def _rest():
            out_ref[0, 0] = out_ref[0, 0] + part


@jax.jit
def kernel(output, target):
    tgt3 = target.reshape(_NR, _BR, 1)
    out = pl.pallas_call(
        _loss_kernel,
        grid=(_NR, _NV),
        in_specs=[
            pl.BlockSpec((_BR, _BV), lambda i, j: (i, j)),
            pl.BlockSpec((_NR, _BR, 1), lambda i, j: (0, 0, 0)),
        ],
        out_specs=pl.BlockSpec((1, 1), lambda i, j: (0, 0),
                               memory_space=pltpu.SMEM),
        out_shape=jax.ShapeDtypeStruct((1, 1), jnp.float32),
        scratch_shapes=[
            pltpu.VMEM((_BR, 1), jnp.float32),
            pltpu.VMEM((_BR, 1), jnp.float32),
            pltpu.VMEM((_BR, 1), jnp.float32),
            pltpu.VMEM((_BR, 1), jnp.float32),
        ],
    )(output, tgt3)
    return out[0, 0]
